# Initial kernel scaffold; baseline (speedup 1.0000x reference)
import jax
import jax.numpy as jnp
from jax.experimental import pallas as pl


def kernel(feat_gene, edge_index_gene, feat_pro, edge_index_pro, W0, W1, W2):
    raise NotImplementedError("write your pallas kernel here")


# speedup vs baseline: 37.9017x; 37.9017x over previous
import functools

import jax
import jax.numpy as jnp
from jax import lax
from jax.experimental import pallas as pl
from jax.experimental.pallas import tpu as pltpu
from jax.experimental.pallas import tpu_sc as plsc

N = 10000
FEAT = 128
E = 320000
NUM_CORES = 2
TILES = 16
N_PAD = 10240
SLICE = N_PAD // TILES
CHUNK = 128
E_TILE = E // TILES
CHUNKS = -(-E_TILE // CHUNK)
E_TILE_PAD = CHUNKS * CHUNK
LAYERS = 3


def _sc_spmm3(feat_g, col_g, row_g, feat_p, col_p, row_p):
    mesh = plsc.VectorSubcoreMesh(
        core_axis_name="c", subcore_axis_name="s", num_cores=NUM_CORES)

    @functools.partial(
        pl.kernel,
        out_type=(jax.ShapeDtypeStruct((N_PAD,), jnp.float32),
                  jax.ShapeDtypeStruct((N_PAD,), jnp.float32)),
        mesh=mesh,
        scratch_types=[
            pltpu.VMEM((CHUNKS, CHUNK), jnp.int32),
            pltpu.VMEM((CHUNKS, CHUNK), jnp.int32),
            pltpu.VMEM((N_PAD,), jnp.float32),
            pltpu.VMEM((CHUNK,), jnp.float32),
            pltpu.VMEM((SLICE,), jnp.float32),
            pltpu.VMEM_SHARED((N_PAD,), jnp.float32),
            pltpu.VMEM_SHARED((N_PAD,), jnp.float32),
        ],
        compiler_params=pltpu.CompilerParams(needs_layout_passes=False),
    )
    def spmm3(fg_hbm, cg_hbm, rg_hbm, fp_hbm, cp_hbm, rp_hbm,
              outg_hbm, outp_hbm,
              col_v, row_v, x_v, vals_v, zeros_v, buf0, buf1):
        cid = lax.axis_index("c")
        sid = lax.axis_index("s")

        def run(feat_hbm, c_hbm, r_hbm, out_hbm):
            pltpu.sync_copy(c_hbm.at[sid], col_v)
            pltpu.sync_copy(r_hbm.at[sid], row_v)
            zvec = jnp.zeros((16,), jnp.float32)
            for j in range(SLICE // 16):
                zeros_v[pl.ds(j * 16, 16)] = zvec

            for layer in range(LAYERS):
                acc = buf0 if layer % 2 == 0 else buf1
                if layer == 0:
                    pltpu.sync_copy(feat_hbm, x_v)
                else:
                    prev = buf1 if layer % 2 == 0 else buf0
                    pltpu.sync_copy(prev, x_v)
                pltpu.sync_copy(zeros_v, acc.at[pl.ds(sid * SLICE, SLICE)])
                plsc.subcore_barrier()

                def chunk_body(j, carry):
                    for v in range(CHUNK // 16):
                        ci = col_v[j, pl.ds(v * 16, 16)]
                        vals_v[pl.ds(v * 16, 16)] = plsc.load_gather(x_v, [ci])
                    pltpu.sync_copy(vals_v, acc.at[row_v.at[j]], add=True)
                    return carry

                lax.fori_loop(0, CHUNKS, chunk_body, 0)
                plsc.subcore_barrier()

            pltpu.sync_copy(buf0.at[pl.ds(sid * SLICE, SLICE)],
                            out_hbm.at[pl.ds(sid * SLICE, SLICE)])

        @pl.when(cid == 0)
        def _():
            run(fg_hbm, cg_hbm, rg_hbm, outg_hbm)

        @pl.when(cid == 1)
        def _():
            run(fp_hbm, cp_hbm, rp_hbm, outp_hbm)

    return spmm3(feat_g, col_g, row_g, feat_p, col_p, row_p)


def _tc_outer(sg, sp, W0, W1, W2):
    def body(sg_ref, sp_ref, w0_ref, w1_ref, w2_ref, og_ref, op_ref):
        w01 = jnp.dot(w0_ref[...], w1_ref[...],
                      preferred_element_type=jnp.float32,
                      precision=lax.Precision.HIGHEST)
        wc = jnp.dot(w01, w2_ref[...],
                     preferred_element_type=jnp.float32,
                     precision=lax.Precision.HIGHEST)
        og_ref[...] = sg_ref[...] * wc
        op_ref[...] = sp_ref[...] * wc

    return pl.pallas_call(
        body,
        out_shape=(jax.ShapeDtypeStruct((N, FEAT), jnp.float32),
                   jax.ShapeDtypeStruct((N, FEAT), jnp.float32)),
    )(sg, sp, W0, W1, W2)


def _prep_edges(edge_index):
    row = edge_index[0].reshape(TILES, E_TILE)
    col = edge_index[1].reshape(TILES, E_TILE)
    pad = E_TILE_PAD - E_TILE
    col = jnp.concatenate(
        [col, jnp.zeros((TILES, pad), jnp.int32)], axis=1)
    row = jnp.concatenate(
        [row, jnp.full((TILES, pad), N, jnp.int32)], axis=1)
    return (col.reshape(TILES, CHUNKS, CHUNK),
            row.reshape(TILES, CHUNKS, CHUNK))


def kernel(feat_gene, edge_index_gene, feat_pro, edge_index_pro, W0, W1, W2):
    fg = jnp.zeros((N_PAD,), jnp.float32).at[:N].set(feat_gene[:, 0])
    fp = jnp.zeros((N_PAD,), jnp.float32).at[:N].set(feat_pro[:, 0])
    col_g, row_g = _prep_edges(edge_index_gene)
    col_p, row_p = _prep_edges(edge_index_pro)
    sg, sp = _sc_spmm3(fg, col_g, row_g, fp, col_p, row_p)
    return _tc_outer(sg[:N, None], sp[:N, None], W0, W1, W2)


# speedup vs baseline: 59.7275x; 1.5758x over previous
import functools

import jax
import jax.numpy as jnp
from jax import lax
from jax.experimental import pallas as pl
from jax.experimental.pallas import tpu as pltpu
from jax.experimental.pallas import tpu_sc as plsc

N = 10000
FEAT = 128
E = 320000
NUM_CORES = 2
TILES = 16
LANES = 16
N_PAD = 10240
SLICE = N_PAD // TILES
E_TILE = E // TILES
UNROLL = 10
GROUPS = E_TILE // LANES
ELOOP_ITERS = GROUPS // UNROLL
LAYERS = 3


def _sc_spmm3(feat_g, row_g, col_g, feat_p, row_p, col_p):
    mesh = plsc.VectorSubcoreMesh(
        core_axis_name="c", subcore_axis_name="s", num_cores=NUM_CORES)

    @functools.partial(
        pl.kernel,
        out_type=(jax.ShapeDtypeStruct((N_PAD,), jnp.float32),
                  jax.ShapeDtypeStruct((N_PAD,), jnp.float32)),
        mesh=mesh,
        scratch_types=[
            pltpu.VMEM((E_TILE,), jnp.int32),
            pltpu.VMEM((E_TILE,), jnp.int32),
            pltpu.VMEM((N_PAD,), jnp.float32),
            pltpu.VMEM((N_PAD,), jnp.float32),
            pltpu.VMEM((SLICE,), jnp.float32),
            pltpu.VMEM((TILES, SLICE), jnp.float32),
            pltpu.VMEM_SHARED((TILES, N_PAD), jnp.float32),
            pltpu.VMEM_SHARED((N_PAD,), jnp.float32),
        ],
        compiler_params=pltpu.CompilerParams(needs_layout_passes=False),
    )
    def spmm3(fg_hbm, rg_hbm, cg_hbm, fp_hbm, rp_hbm, cp_hbm,
              outg_hbm, outp_hbm,
              row_v, col_v, x_v, acc_v, red_v, tmp_v, stage, xshare):
        cid = lax.axis_index("c")
        sid = lax.axis_index("s")

        def zero_acc():
            zvec = jnp.zeros((LANES,), jnp.float32)

            def zloop(i, c):
                for u in range(8):
                    acc_v[pl.ds((i * 8 + u) * LANES, LANES)] = zvec
                return c

            lax.fori_loop(0, N_PAD // LANES // 8, zloop, 0)

        def run(feat_hbm, r_hbm, c_hbm, out_hbm):
            base = sid * E_TILE
            pltpu.sync_copy(r_hbm.at[pl.ds(base, E_TILE)], row_v)
            pltpu.sync_copy(c_hbm.at[pl.ds(base, E_TILE)], col_v)
            zero_acc()
            pltpu.sync_copy(feat_hbm, x_v)
            myoff = sid * SLICE

            for layer in range(LAYERS):
                def eloop(i, c):
                    gbase = i * (LANES * UNROLL)
                    for u in range(UNROLL):
                        off = gbase + u * LANES
                        ci = col_v[pl.ds(off, LANES)]
                        ri = row_v[pl.ds(off, LANES)]
                        vals = plsc.load_gather(x_v, [ci])
                        plsc.addupdate_scatter(acc_v, [ri], vals)
                    return c

                lax.fori_loop(0, ELOOP_ITERS, eloop, 0)

                pltpu.sync_copy(acc_v, stage.at[sid])
                if layer < LAYERS - 1:
                    zero_acc()
                plsc.subcore_barrier()

                pltpu.sync_copy(stage.at[:, pl.ds(myoff, SLICE)], tmp_v)

                def rloop(j, c):
                    off = j * LANES
                    s = tmp_v[0, pl.ds(off, LANES)]
                    for t in range(1, TILES):
                        s = s + tmp_v[t, pl.ds(off, LANES)]
                    red_v[pl.ds(off, LANES)] = s
                    return c

                lax.fori_loop(0, SLICE // LANES, rloop, 0)

                if layer < LAYERS - 1:
                    pltpu.sync_copy(red_v, xshare.at[pl.ds(myoff, SLICE)])
                    plsc.subcore_barrier()
                    pltpu.sync_copy(xshare, x_v)
                else:
                    pltpu.sync_copy(red_v, out_hbm.at[pl.ds(myoff, SLICE)])

        @pl.when(cid == 0)
        def _():
            run(fg_hbm, rg_hbm, cg_hbm, outg_hbm)

        @pl.when(cid == 1)
        def _():
            run(fp_hbm, rp_hbm, cp_hbm, outp_hbm)

    return spmm3(feat_g, row_g, col_g, feat_p, row_p, col_p)


def _tc_outer(sg, sp, W0, W1, W2):
    def body(sg_ref, sp_ref, w0_ref, w1_ref, w2_ref, og_ref, op_ref):
        w01 = jnp.dot(w0_ref[...], w1_ref[...],
                      preferred_element_type=jnp.float32,
                      precision=lax.Precision.HIGHEST)
        wc = jnp.dot(w01, w2_ref[...],
                     preferred_element_type=jnp.float32,
                     precision=lax.Precision.HIGHEST)
        og_ref[...] = sg_ref[...] * wc
        op_ref[...] = sp_ref[...] * wc

    return pl.pallas_call(
        body,
        out_shape=(jax.ShapeDtypeStruct((N, FEAT), jnp.float32),
                   jax.ShapeDtypeStruct((N, FEAT), jnp.float32)),
    )(sg, sp, W0, W1, W2)


def kernel(feat_gene, edge_index_gene, feat_pro, edge_index_pro, W0, W1, W2):
    fg = jnp.pad(feat_gene[:, 0], (0, N_PAD - N))
    fp = jnp.pad(feat_pro[:, 0], (0, N_PAD - N))
    sg, sp = _sc_spmm3(fg, edge_index_gene[0], edge_index_gene[1],
                       fp, edge_index_pro[0], edge_index_pro[1])
    return _tc_outer(sg[:N, None], sp[:N, None], W0, W1, W2)


# speedup vs baseline: 81.4763x; 1.3641x over previous
import functools

import jax
import jax.numpy as jnp
from jax import lax
from jax.experimental import pallas as pl
from jax.experimental.pallas import tpu as pltpu
from jax.experimental.pallas import tpu_sc as plsc

N = 10000
FEAT = 128
E = 320000
NUM_CORES = 2
TILES = 16
LANES = 16
N_PAD = 10240
SLICE = N_PAD // TILES
UNROLL = 8
EGRP = LANES * UNROLL
COL_TILES = E // 128
HI_SUBCORES = 4
G_HI = 157
G_LO = 156
E_TILE_HI = G_HI * 128
E_TILE_LO = G_LO * 128
LAYERS = 3
ROW_BLK = 1024
OUT_GRID = (N + ROW_BLK - 1) // ROW_BLK
SUB = ROW_BLK // FEAT


def _sc_spmm3(feat_g, eidx_g, feat_p, eidx_p):
    mesh = plsc.VectorSubcoreMesh(
        core_axis_name="c", subcore_axis_name="s", num_cores=NUM_CORES)

    @functools.partial(
        pl.kernel,
        out_type=(jax.ShapeDtypeStruct((N_PAD,), jnp.float32),
                  jax.ShapeDtypeStruct((N_PAD,), jnp.float32)),
        mesh=mesh,
        scratch_types=[
            pltpu.VMEM((2, E_TILE_HI), jnp.int32),
            pltpu.VMEM((N_PAD,), jnp.float32),
            pltpu.VMEM((N_PAD,), jnp.float32),
            pltpu.VMEM((SLICE,), jnp.float32),
            pltpu.VMEM((TILES, SLICE), jnp.float32),
            pltpu.VMEM_SHARED((TILES, N_PAD), jnp.float32),
            pltpu.VMEM_SHARED((N_PAD,), jnp.float32),
        ],
        compiler_params=pltpu.CompilerParams(needs_layout_passes=False),
    )
    def spmm3(fg_hbm, eg_hbm, fp_hbm, ep_hbm,
              outg_hbm, outp_hbm,
              e_v, x_v, acc_v, red_v, tmp_v, stage, xshare):
        cid = lax.axis_index("c")
        sid = lax.axis_index("s")

        def zero_acc():
            zvec = jnp.zeros((LANES,), jnp.float32)

            def zloop(i, c):
                for u in range(8):
                    acc_v[pl.ds((i * 8 + u) * LANES, LANES)] = zvec
                return c

            lax.fori_loop(0, N_PAD // LANES // 8, zloop, 0)

        def run(feat_hbm, e_hbm, out_hbm):
            @pl.when(sid < HI_SUBCORES)
            def _():
                pltpu.sync_copy(
                    e_hbm.at[:, pl.ds(sid * E_TILE_HI, E_TILE_HI)], e_v)

            @pl.when(sid >= HI_SUBCORES)
            def _():
                start = (HI_SUBCORES * E_TILE_HI
                         + (sid - HI_SUBCORES) * E_TILE_LO)
                pltpu.sync_copy(e_hbm.at[:, pl.ds(start, E_TILE_LO)],
                                e_v.at[:, pl.ds(0, E_TILE_LO)])

            eiters = jnp.where(sid < HI_SUBCORES, G_HI, G_LO)
            zero_acc()
            pltpu.sync_copy(feat_hbm, x_v.at[pl.ds(0, N)])
            myoff = sid * SLICE

            for layer in range(LAYERS):
                def eloop(i, c):
                    gbase = i * EGRP
                    for u in range(UNROLL):
                        off = gbase + u * LANES
                        ci = e_v[1, pl.ds(off, LANES)]
                        ri = e_v[0, pl.ds(off, LANES)]
                        vals = plsc.load_gather(x_v, [ci])
                        plsc.addupdate_scatter(acc_v, [ri], vals)
                    return c

                lax.fori_loop(0, eiters, eloop, 0)

                pltpu.sync_copy(acc_v, stage.at[sid])
                if layer < LAYERS - 1:
                    zero_acc()
                plsc.subcore_barrier()

                pltpu.sync_copy(stage.at[:, pl.ds(myoff, SLICE)], tmp_v)

                def rloop(j, c):
                    off = j * LANES
                    s = tmp_v[0, pl.ds(off, LANES)]
                    for t in range(1, TILES):
                        s = s + tmp_v[t, pl.ds(off, LANES)]
                    red_v[pl.ds(off, LANES)] = s
                    return c

                lax.fori_loop(0, SLICE // LANES, rloop, 0)

                if layer < LAYERS - 1:
                    pltpu.sync_copy(red_v, xshare.at[pl.ds(myoff, SLICE)])
                    plsc.subcore_barrier()
                    pltpu.sync_copy(xshare, x_v)
                else:
                    pltpu.sync_copy(red_v, out_hbm.at[pl.ds(myoff, SLICE)])

        @pl.when(cid == 0)
        def _():
            run(fg_hbm, eg_hbm, outg_hbm)

        @pl.when(cid == 1)
        def _():
            run(fp_hbm, ep_hbm, outp_hbm)

    return spmm3(feat_g, eidx_g, feat_p, eidx_p)


def _tc_wc(W0, W1, W2):
    def body(w0_ref, w1_ref, w2_ref, wc_ref):
        w01 = jnp.dot(w0_ref[...], w1_ref[...],
                      preferred_element_type=jnp.float32,
                      precision=lax.Precision.HIGHEST)
        wc_ref[...] = jnp.dot(w01, w2_ref[...],
                              preferred_element_type=jnp.float32,
                              precision=lax.Precision.HIGHEST)

    return pl.pallas_call(
        body,
        out_shape=jax.ShapeDtypeStruct((1, FEAT), jnp.float32),
    )(W0, W1, W2)


def _tc_outer(sg2d, sp2d, wc):
    contract = (((0,), (0,)), ((), ()))

    def body(sg_ref, sp_ref, wc_ref, og_ref, op_ref):
        w = wc_ref[...]
        for q in range(SUB):
            og_ref[q * FEAT:(q + 1) * FEAT, :] = lax.dot_general(
                sg_ref[q:q + 1, :], w, contract,
                preferred_element_type=jnp.float32)
            op_ref[q * FEAT:(q + 1) * FEAT, :] = lax.dot_general(
                sp_ref[q:q + 1, :], w, contract,
                preferred_element_type=jnp.float32)

    return pl.pallas_call(
        body,
        grid=(OUT_GRID,),
        in_specs=[
            pl.BlockSpec((SUB, FEAT), lambda i: (i, 0)),
            pl.BlockSpec((SUB, FEAT), lambda i: (i, 0)),
            pl.BlockSpec((1, FEAT), lambda i: (0, 0)),
        ],
        out_specs=[
            pl.BlockSpec((ROW_BLK, FEAT), lambda i: (i, 0)),
            pl.BlockSpec((ROW_BLK, FEAT), lambda i: (i, 0)),
        ],
        out_shape=(jax.ShapeDtypeStruct((N, FEAT), jnp.float32),
                   jax.ShapeDtypeStruct((N, FEAT), jnp.float32)),
    )(sg2d, sp2d, wc)


def kernel(feat_gene, edge_index_gene, feat_pro, edge_index_pro, W0, W1, W2):
    fg = feat_gene.reshape(N)
    fp = feat_pro.reshape(N)
    wc = _tc_wc(W0, W1, W2)
    sg, sp = _sc_spmm3(fg, edge_index_gene, fp, edge_index_pro)
    return _tc_outer(sg.reshape(N_PAD // FEAT, FEAT),
                     sp.reshape(N_PAD // FEAT, FEAT), wc)
